# baseline (speedup 1.0000x reference)
import functools

import jax
import jax.numpy as jnp
from jax import lax
from jax.experimental import pallas as pl
from jax.experimental.pallas import tpu as pltpu
from jax.experimental.pallas import tpu_sc as plsc

B, C, H, W = 16, 96, 224, 224
E, K, RED = 16, 2, 6
HO, WO = H // 2, W // 2
CBLK = 48
NCB = C // CBLK


def _silu(v):
    return v / (1.0 + jnp.exp(-v))


def _logits_body(x_ref, a_ref, shift_ref, pw_ref, sel_ref, lwt_ref, lb_ref,
                 out_ref, t_ref):
    cb = pl.program_id(1)
    xb = x_ref[0]
    zrow = jnp.zeros((CBLK, 1, 2 * W), jnp.float32)
    xb_up = jnp.concatenate([zrow, xb[:, :HO - 1, :]], axis=1)

    def v(dj):
        t = (a_ref[:, dj:dj + 1, :] * xb
             + a_ref[:, 3 + dj:4 + dj, :] * xb_up)
        return t.astype(jnp.bfloat16).reshape(CBLK * HO, 2 * W)

    def dec(vd, dj):
        return jax.lax.dot_general(
            vd, sel_ref[dj], (((1,), (0,)), ((), ())),
            preferred_element_type=jnp.float32)

    ydec = dec(v(0), 0) + dec(v(1), 1) + dec(v(2), 2)
    y = ydec.reshape(CBLK, HO, WO) + shift_ref[...]
    s = _silu(y)

    acc = [None] * RED
    for r in range(RED):
        pw_r = pw_ref[:, r:r + 1, :]
        acc[r] = jnp.sum(pw_r * s, axis=0)

    @pl.when(cb == 0)
    def _init_acc():
        for r in range(RED):
            t_ref[r] = acc[r]

    @pl.when(cb > 0)
    def _accum():
        for r in range(RED):
            t_ref[r] += acc[r]

    @pl.when(cb == NCB - 1)
    def _finish():
        res = lb_ref[...]
        for r in range(RED):
            f_r = jnp.sum(_silu(t_ref[r])) * (1.0 / (HO * WO))
            res = res + f_r * lwt_ref[r:r + 1, :]
        out_ref[...] = res[None]


@jax.jit
def _router_logits_tc(x6, a, shift3, pwt, sel3, lwt, lb2):
    return pl.pallas_call(
        _logits_body,
        grid=(B, NCB),
        in_specs=[
            pl.BlockSpec((1, CBLK, HO, 2 * W), lambda b, cb: (b, cb, 0, 0)),
            pl.BlockSpec((CBLK, 6, 2 * W), lambda b, cb: (cb, 0, 0)),
            pl.BlockSpec((CBLK, 1, WO), lambda b, cb: (cb, 0, 0)),
            pl.BlockSpec((CBLK, RED, WO), lambda b, cb: (cb, 0, 0)),
            pl.BlockSpec((3, 2 * W, WO), lambda b, cb: (0, 0, 0)),
            pl.BlockSpec((RED, E), lambda b, cb: (0, 0)),
            pl.BlockSpec((1, E), lambda b, cb: (0, 0)),
        ],
        out_specs=pl.BlockSpec((1, 1, E), lambda b, cb: (b, 0, 0)),
        out_shape=jax.ShapeDtypeStruct((B, 1, E), jnp.float32),
        scratch_shapes=[
            pltpu.VMEM((RED, HO, WO), jnp.float32),
        ],
    )(x6, a, shift3, pwt, sel3, lwt, lb2).reshape(B, E)


def _topk_body(lgt_hbm, out_hbm, lgt_v, out_v):
    c = lax.axis_index("c")
    s = lax.axis_index("s")

    @pl.when((c == 0) & (s == 0))
    def _():
        pltpu.sync_copy(lgt_hbm, lgt_v)
        m1 = lgt_v[0]
        i1 = jnp.zeros((E,), jnp.int32)
        m2 = jnp.full((E,), -jnp.inf, jnp.float32)
        i2 = jnp.zeros((E,), jnp.int32)
        for j in range(1, E):
            v = lgt_v[j]
            jv = jnp.full((E,), j, jnp.int32)
            gt1 = v > m1
            gt2 = v > m2
            i2 = jnp.where(gt1, i1, jnp.where(gt2, jv, i2))
            m2 = jnp.where(gt1, m1, jnp.where(gt2, v, m2))
            i1 = jnp.where(gt1, jv, i1)
            m1 = jnp.where(gt1, v, m1)
        out_v[0] = i1
        out_v[1] = i2
        pltpu.sync_copy(out_v, out_hbm)


@jax.jit
def _topk_sc(lgt):
    mesh = plsc.VectorSubcoreMesh(core_axis_name="c", subcore_axis_name="s")
    fn = functools.partial(
        pl.kernel,
        out_type=jax.ShapeDtypeStruct((K, B), jnp.int32),
        mesh=mesh,
        scratch_types=[
            pltpu.VMEM((E, B), jnp.float32),
            pltpu.VMEM((K, B), jnp.int32),
        ],
    )(_topk_body)
    return fn(lgt)


def _prep(x, dw_w, bn_gamma, bn_beta, bn_mean, bn_var, pw_w, lin_w, lin_b):
    scale = bn_gamma / jnp.sqrt(bn_var + 1e-5)
    shift = bn_beta - bn_mean * scale
    w9 = dw_w.reshape(C, 3, 3) * scale[:, None, None]
    a_lo = jnp.concatenate(
        [jnp.broadcast_to(w9[:, 1, :, None], (C, 3, W)),
         jnp.broadcast_to(w9[:, 2, :, None], (C, 3, W))], axis=2)
    a_hi = jnp.concatenate(
        [jnp.zeros((C, 3, W), jnp.float32),
         jnp.broadcast_to(w9[:, 0, :, None], (C, 3, W))], axis=2)
    a = jnp.concatenate([a_lo, a_hi], axis=1)
    shift3 = jnp.broadcast_to(shift[:, None, None], (C, 1, WO))
    pwt = jnp.broadcast_to(pw_w.reshape(RED, C).T[:, :, None], (C, RED, WO))
    lwt = lin_w.T
    lb2 = lin_b[None, :]
    x6 = x.reshape(B, C, HO, 2 * W)
    j = jnp.arange(W)[None, :, None]
    q = jnp.arange(WO)[None, None, :]
    dj = jnp.arange(3)[:, None, None]
    sel = (j == 2 * q + dj - 1).astype(jnp.bfloat16)
    sel3 = jnp.concatenate([sel, sel], axis=1)
    return x6, a, shift3, pwt, sel3, lwt, lb2


def kernel(x, dw_w, bn_gamma, bn_beta, bn_mean, bn_var, pw_w, lin_w, lin_b):
    args = _prep(x, dw_w, bn_gamma, bn_beta, bn_mean, bn_var, pw_w, lin_w,
                 lin_b)
    logits = _router_logits_tc(*args)
    idx = _topk_sc(logits.T).T
    weights = jnp.ones((B, K), jnp.float32)
    return (weights, idx, logits)
